# baseline (device time: 39573 ns/iter reference)
import os

import jax
import jax.numpy as jnp
from jax import lax
from jax.experimental import pallas as pl
from jax.experimental.pallas import tpu as pltpu

N_DEV = 32
N_STEPS = 5
DH = 64


def kernel(x, Wq, Wo, Wk, Wv):
    B, Sq, D = x.shape
    Hd = Wq.shape[1]
    Hq = Hd // DH
    bf16 = jnp.bfloat16
    n_steps = 0 if os.environ.get("ABLATE_COMM") == "1" else N_STEPS

    def body(x_ref, wq_ref, wo_ref, wk_ref, wv_ref, out_ref,
             o_ref, acc_ref, sbuf_ref, comm_ref, send_sems, recv_sems):
        my_pos = lax.axis_index("i")
        partners = [my_pos ^ (1 << s) for s in range(N_STEPS)]

        if n_steps:
            barrier_sem = pltpu.get_barrier_semaphore()
            for p in partners:
                pl.semaphore_signal(
                    barrier_sem, inc=1, device_id=p,
                    device_id_type=pl.DeviceIdType.LOGICAL,
                )
            pl.semaphore_wait(barrier_sem, N_STEPS)

        def make_rdma(c, s):
            slot = c * N_STEPS + s
            return pltpu.make_async_remote_copy(
                src_ref=sbuf_ref.at[c],
                dst_ref=comm_ref.at[slot],
                send_sem=send_sems.at[slot],
                recv_sem=recv_sems.at[slot],
                device_id=partners[s],
                device_id_type=pl.DeviceIdType.LOGICAL,
            )

        rd = {}

        def compute_batch(b):
            xb = x_ref[b].astype(bf16)
            q = jnp.dot(xb, wq_ref[:].astype(bf16),
                        preferred_element_type=jnp.float32).astype(bf16)
            k = jnp.dot(xb, wk_ref[:].astype(bf16),
                        preferred_element_type=jnp.float32).astype(bf16)
            v = jnp.dot(xb, wv_ref[:].astype(bf16),
                        preferred_element_type=jnp.float32).astype(bf16)
            for h in range(Hq):
                cols = slice(h * DH, (h + 1) * DH)
                s_ = lax.dot_general(
                    q[:, cols], k[:, cols], (((1,), (1,)), ((), ())),
                    preferred_element_type=jnp.float32,
                ) * 0.125
                m = jnp.max(s_, axis=-1, keepdims=True)
                p_ = jnp.exp(s_ - m)
                l_ = jnp.sum(p_, axis=-1, keepdims=True)
                pn = (p_ / l_).astype(bf16)
                o_ref[b, :, cols] = jnp.dot(
                    pn, v[:, cols], preferred_element_type=jnp.float32
                ).astype(bf16)
            acc_ref[b] = jnp.dot(
                o_ref[b], wo_ref[:].astype(bf16),
                preferred_element_type=jnp.float32,
            )

        def launch(c, s):
            sbuf_ref[c] = acc_ref[c].astype(bf16)
            rd[(c, s)] = make_rdma(c, s)
            rd[(c, s)].start()

        compute_batch(0)
        if n_steps:
            launch(0, 0)
        compute_batch(1)
        if n_steps:
            launch(1, 0)

        for s in range(n_steps):
            for c in range(2):
                rd[(c, s)].wait()
                acc_ref[c] = acc_ref[c] + comm_ref[c * N_STEPS + s].astype(
                    jnp.float32)
                if s + 1 < n_steps:
                    launch(c, s + 1)

        out_ref[:] = acc_ref[:]

    return pl.pallas_call(
        body,
        out_shape=jax.ShapeDtypeStruct((B, Sq, D), jnp.float32),
        in_specs=[pl.BlockSpec(memory_space=pltpu.VMEM)] * 5,
        out_specs=pl.BlockSpec(memory_space=pltpu.VMEM),
        scratch_shapes=[
            pltpu.VMEM((B, Sq, Hd), bf16),
            pltpu.VMEM((B, Sq, D), jnp.float32),
            pltpu.VMEM((B, Sq, D), bf16),
            pltpu.VMEM((B * N_STEPS, Sq, D), bf16),
            pltpu.SemaphoreType.DMA((B * N_STEPS,)),
            pltpu.SemaphoreType.DMA((B * N_STEPS,)),
        ],
        compiler_params=pltpu.CompilerParams(collective_id=0),
    )(x, Wq, Wo, Wk, Wv)


# device time: 39373 ns/iter; 1.0051x vs baseline; 1.0051x over previous
import os

import jax
import jax.numpy as jnp
from jax import lax
from jax.experimental import pallas as pl
from jax.experimental.pallas import tpu as pltpu

N_DEV = 32
N_STEPS = 5
DH = 64


def kernel(x, Wq, Wo, Wk, Wv):
    B, Sq, D = x.shape
    Hd = Wq.shape[1]
    Hq = Hd // DH
    bf16 = jnp.bfloat16
    n_steps = 0 if os.environ.get("ABLATE_COMM") == "1" else N_STEPS

    def body(x_ref, wq_ref, wo_ref, wk_ref, wv_ref, out_ref,
             o_ref, acc_ref, sbuf_ref, comm_ref, send_sems, recv_sems):
        my_pos = lax.axis_index("i")
        partners = [my_pos ^ (1 << s) for s in range(N_STEPS)]

        if n_steps:
            barrier_sem = pltpu.get_barrier_semaphore()
            for p in partners:
                pl.semaphore_signal(
                    barrier_sem, inc=1, device_id=p,
                    device_id_type=pl.DeviceIdType.LOGICAL,
                )
            pl.semaphore_wait(barrier_sem, N_STEPS)

        n_chunks = 4
        hs = Sq // 2

        def make_rdma(c, s):
            slot = c * N_STEPS + s
            return pltpu.make_async_remote_copy(
                src_ref=sbuf_ref.at[c],
                dst_ref=comm_ref.at[slot],
                send_sem=send_sems.at[slot],
                recv_sem=recv_sems.at[slot],
                device_id=partners[s],
                device_id_type=pl.DeviceIdType.LOGICAL,
            )

        rd = {}

        def compute_batch(b):
            xb = x_ref[b].astype(bf16)
            q = jnp.dot(xb, wq_ref[:].astype(bf16),
                        preferred_element_type=jnp.float32).astype(bf16)
            k = jnp.dot(xb, wk_ref[:].astype(bf16),
                        preferred_element_type=jnp.float32).astype(bf16)
            v = jnp.dot(xb, wv_ref[:].astype(bf16),
                        preferred_element_type=jnp.float32).astype(bf16)
            for h in range(Hq):
                cols = slice(h * DH, (h + 1) * DH)
                s_ = lax.dot_general(
                    q[:, cols], k[:, cols], (((1,), (1,)), ((), ())),
                    preferred_element_type=jnp.float32,
                ) * 0.125
                m = jnp.max(s_, axis=-1, keepdims=True)
                p_ = jnp.exp(s_ - m)
                l_ = jnp.sum(p_, axis=-1, keepdims=True)
                pn = (p_ / l_).astype(bf16)
                o_ref[b, :, cols] = jnp.dot(
                    pn, v[:, cols], preferred_element_type=jnp.float32
                ).astype(bf16)
            acc_ref[b] = jnp.dot(
                o_ref[b], wo_ref[:].astype(bf16),
                preferred_element_type=jnp.float32,
            )

        def launch(c, s):
            b, rh = divmod(c, 2)
            rows = slice(rh * hs, (rh + 1) * hs)
            sbuf_ref[c] = acc_ref[b, rows].astype(bf16)
            rd[(c, s)] = make_rdma(c, s)
            rd[(c, s)].start()

        compute_batch(0)
        if n_steps:
            launch(0, 0)
            launch(1, 0)
        compute_batch(1)
        if n_steps:
            launch(2, 0)
            launch(3, 0)

        for s in range(n_steps):
            for c in range(n_chunks):
                b, rh = divmod(c, 2)
                rows = slice(rh * hs, (rh + 1) * hs)
                rd[(c, s)].wait()
                acc_ref[b, rows] = acc_ref[b, rows] + comm_ref[
                    c * N_STEPS + s].astype(jnp.float32)
                if s + 1 < n_steps:
                    launch(c, s + 1)

        out_ref[:] = acc_ref[:]

    return pl.pallas_call(
        body,
        out_shape=jax.ShapeDtypeStruct((B, Sq, D), jnp.float32),
        in_specs=[pl.BlockSpec(memory_space=pltpu.VMEM)] * 5,
        out_specs=pl.BlockSpec(memory_space=pltpu.VMEM),
        scratch_shapes=[
            pltpu.VMEM((B, Sq, Hd), bf16),
            pltpu.VMEM((B, Sq, D), jnp.float32),
            pltpu.VMEM((4, Sq // 2, D), bf16),
            pltpu.VMEM((4 * N_STEPS, Sq // 2, D), bf16),
            pltpu.SemaphoreType.DMA((4 * N_STEPS,)),
            pltpu.SemaphoreType.DMA((4 * N_STEPS,)),
        ],
        compiler_params=pltpu.CompilerParams(collective_id=0),
    )(x, Wq, Wo, Wk, Wv)


# device time: 27225 ns/iter; 1.4536x vs baseline; 1.4462x over previous
import os

import jax
import jax.numpy as jnp
from jax import lax
from jax.experimental import pallas as pl
from jax.experimental.pallas import tpu as pltpu

N_DEV = 32
DH = 64


def kernel(x, Wq, Wo, Wk, Wv):
    B, Sq, D = x.shape
    Hd = Wq.shape[1]
    Hq = Hd // DH
    R = B * Sq
    P = R // N_DEV
    bf16 = jnp.bfloat16
    comm_on = os.environ.get("ABLATE_COMM") != "1"

    def body(x_ref, wq_ref, wo_ref, wk_ref, wv_ref, out_ref,
             o_ref, acc_ref, sbuf_ref, rs_ref, agbuf_ref, ag_ref,
             rs_send_sem, rs_recv_sem, ag_send_sem, ag_recv_sem):
        my_pos = lax.axis_index("i")

        if comm_on:
            barrier_sem = pltpu.get_barrier_semaphore()
            for j in range(N_DEV):
                pl.semaphore_signal(
                    barrier_sem, inc=1, device_id=j,
                    device_id_type=pl.DeviceIdType.LOGICAL,
                )

        def compute_batch(b):
            xb = x_ref[b].astype(bf16)
            q = jnp.dot(xb, wq_ref[:].astype(bf16),
                        preferred_element_type=jnp.float32).astype(bf16)
            k = jnp.dot(xb, wk_ref[:].astype(bf16),
                        preferred_element_type=jnp.float32).astype(bf16)
            v = jnp.dot(xb, wv_ref[:].astype(bf16),
                        preferred_element_type=jnp.float32).astype(bf16)
            for h in range(Hq):
                cols = slice(h * DH, (h + 1) * DH)
                s_ = lax.dot_general(
                    q[:, cols], k[:, cols], (((1,), (1,)), ((), ())),
                    preferred_element_type=jnp.float32,
                ) * 0.125
                m = jnp.max(s_, axis=-1, keepdims=True)
                p_ = jnp.exp(s_ - m)
                l_ = jnp.sum(p_, axis=-1, keepdims=True)
                pn = (p_ / l_).astype(bf16)
                o_ref[b, :, cols] = jnp.dot(
                    pn, v[:, cols], preferred_element_type=jnp.float32
                ).astype(bf16)
            acc_ref[b] = jnp.dot(
                o_ref[b], wo_ref[:].astype(bf16),
                preferred_element_type=jnp.float32,
            )

        def rs_send(j):
            return pltpu.make_async_remote_copy(
                src_ref=sbuf_ref.at[j],
                dst_ref=rs_ref.at[my_pos],
                send_sem=rs_send_sem,
                recv_sem=rs_recv_sem,
                device_id=j,
                device_id_type=pl.DeviceIdType.LOGICAL,
            )

        compute_batch(0)
        if comm_on:
            pl.semaphore_wait(barrier_sem, N_DEV)
            sbuf_ref[0:N_DEV // 2] = acc_ref[0].astype(bf16).reshape(
                N_DEV // 2, P, D)
            for j in range(N_DEV // 2):
                rs_send(j).start()
        compute_batch(1)

        if not comm_on:
            out_ref[:] = acc_ref[:]
            return

        sbuf_ref[N_DEV // 2:N_DEV] = acc_ref[1].astype(bf16).reshape(
            N_DEV // 2, P, D)
        for j in range(N_DEV // 2, N_DEV):
            rs_send(j).start()

        for _ in range(N_DEV):
            rs_send(0).wait_recv()
        agbuf_ref[:] = jnp.sum(
            rs_ref[:].astype(jnp.float32), axis=0).astype(bf16)

        def ag_send(j):
            return pltpu.make_async_remote_copy(
                src_ref=agbuf_ref,
                dst_ref=ag_ref.at[my_pos],
                send_sem=ag_send_sem,
                recv_sem=ag_recv_sem,
                device_id=j,
                device_id_type=pl.DeviceIdType.LOGICAL,
            )

        for j in range(N_DEV):
            ag_send(j).start()
        for _ in range(N_DEV):
            ag_send(0).wait_recv()

        out_ref[:] = ag_ref[:].astype(jnp.float32).reshape(B, Sq, D)

        for _ in range(N_DEV):
            rs_send(0).wait_send()
            ag_send(0).wait_send()

    return pl.pallas_call(
        body,
        out_shape=jax.ShapeDtypeStruct((B, Sq, D), jnp.float32),
        in_specs=[pl.BlockSpec(memory_space=pltpu.VMEM)] * 5,
        out_specs=pl.BlockSpec(memory_space=pltpu.VMEM),
        scratch_shapes=[
            pltpu.VMEM((B, Sq, Hd), bf16),
            pltpu.VMEM((B, Sq, D), jnp.float32),
            pltpu.VMEM((N_DEV, P, D), bf16),
            pltpu.VMEM((N_DEV, P, D), bf16),
            pltpu.VMEM((P, D), bf16),
            pltpu.VMEM((N_DEV, P, D), bf16),
            pltpu.SemaphoreType.DMA,
            pltpu.SemaphoreType.DMA,
            pltpu.SemaphoreType.DMA,
            pltpu.SemaphoreType.DMA,
        ],
        compiler_params=pltpu.CompilerParams(collective_id=0),
    )(x, Wq, Wo, Wk, Wv)


# device time: 23719 ns/iter; 1.6684x vs baseline; 1.1478x over previous
import os

import jax
import jax.numpy as jnp
from jax import lax
from jax.experimental import pallas as pl
from jax.experimental.pallas import tpu as pltpu

N_DEV = 32
DH = 64


def kernel(x, Wq, Wo, Wk, Wv):
    B, Sq, D = x.shape
    Hd = Wq.shape[1]
    Hq = Hd // DH
    R = B * Sq
    P = R // N_DEV
    bf16 = jnp.bfloat16
    comm_on = os.environ.get("ABLATE_COMM") != "1"

    def body(x_ref, wq_ref, wo_ref, wk_ref, wv_ref, out_ref,
             o_ref, acc_ref, s_ref, sbuf_ref, rs_ref, agbuf_ref, ag_ref,
             rs_send_sem, rs_recv_sem, ag_send_sem, ag_recv_sem):
        my_pos = lax.axis_index("i")

        if comm_on:
            barrier_sem = pltpu.get_barrier_semaphore()
            for j in range(N_DEV):
                pl.semaphore_signal(
                    barrier_sem, inc=1, device_id=j,
                    device_id_type=pl.DeviceIdType.LOGICAL,
                )

        x2 = x_ref[:].reshape(R, D).astype(bf16)
        qf = jnp.dot(x2, wq_ref[:].astype(bf16),
                     preferred_element_type=jnp.float32).astype(bf16)
        kf = jnp.dot(x2, wk_ref[:].astype(bf16),
                     preferred_element_type=jnp.float32).astype(bf16)
        vf = jnp.dot(x2, wv_ref[:].astype(bf16),
                     preferred_element_type=jnp.float32).astype(bf16)

        def compute_batch(b):
            rows = slice(b * Sq, (b + 1) * Sq)
            for h in range(Hq):
                cols = slice(h * DH, (h + 1) * DH)
                s_ref[h] = lax.dot_general(
                    qf[rows, cols], kf[rows, cols], (((1,), (1,)), ((), ())),
                    preferred_element_type=jnp.float32,
                )
            s_all = s_ref[:] * 0.125
            m = jnp.max(s_all, axis=-1, keepdims=True)
            p_ = jnp.exp(s_all - m)
            l_ = jnp.sum(p_, axis=-1, keepdims=True)
            pn = (p_ / l_).astype(bf16)
            for h in range(Hq):
                cols = slice(h * DH, (h + 1) * DH)
                o_ref[b, :, cols] = jnp.dot(
                    pn[h], vf[rows, cols], preferred_element_type=jnp.float32
                ).astype(bf16)
            acc_ref[b] = jnp.dot(
                o_ref[b], wo_ref[:].astype(bf16),
                preferred_element_type=jnp.float32,
            )

        def rs_send(j):
            return pltpu.make_async_remote_copy(
                src_ref=sbuf_ref.at[j],
                dst_ref=rs_ref.at[my_pos],
                send_sem=rs_send_sem,
                recv_sem=rs_recv_sem,
                device_id=j,
                device_id_type=pl.DeviceIdType.LOGICAL,
            )

        compute_batch(0)
        if comm_on:
            pl.semaphore_wait(barrier_sem, N_DEV)
            sbuf_ref[0:N_DEV // 2] = acc_ref[0].astype(bf16).reshape(
                N_DEV // 2, P, D)
            for j in range(N_DEV // 2):
                rs_send(j).start()
        compute_batch(1)

        if not comm_on:
            out_ref[:] = acc_ref[:]
            return

        sbuf_ref[N_DEV // 2:N_DEV] = acc_ref[1].astype(bf16).reshape(
            N_DEV // 2, P, D)
        for j in range(N_DEV // 2, N_DEV):
            rs_send(j).start()

        for _ in range(N_DEV):
            rs_send(0).wait_recv()
        agbuf_ref[:] = jnp.sum(
            rs_ref[:].astype(jnp.float32), axis=0).astype(bf16)

        def ag_send(j):
            return pltpu.make_async_remote_copy(
                src_ref=agbuf_ref,
                dst_ref=ag_ref.at[my_pos],
                send_sem=ag_send_sem,
                recv_sem=ag_recv_sem,
                device_id=j,
                device_id_type=pl.DeviceIdType.LOGICAL,
            )

        for j in range(N_DEV):
            ag_send(j).start()
        for _ in range(N_DEV):
            ag_send(0).wait_recv()

        out_ref[:] = ag_ref[:].astype(jnp.float32).reshape(B, Sq, D)

        for _ in range(N_DEV):
            rs_send(0).wait_send()
            ag_send(0).wait_send()

    return pl.pallas_call(
        body,
        out_shape=jax.ShapeDtypeStruct((B, Sq, D), jnp.float32),
        in_specs=[pl.BlockSpec(memory_space=pltpu.VMEM)] * 5,
        out_specs=pl.BlockSpec(memory_space=pltpu.VMEM),
        scratch_shapes=[
            pltpu.VMEM((B, Sq, Hd), bf16),
            pltpu.VMEM((B, Sq, D), jnp.float32),
            pltpu.VMEM((Hq, Sq, Sq), jnp.float32),
            pltpu.VMEM((N_DEV, P, D), bf16),
            pltpu.VMEM((N_DEV, P, D), bf16),
            pltpu.VMEM((P, D), bf16),
            pltpu.VMEM((N_DEV, P, D), bf16),
            pltpu.SemaphoreType.DMA,
            pltpu.SemaphoreType.DMA,
            pltpu.SemaphoreType.DMA,
            pltpu.SemaphoreType.DMA,
        ],
        compiler_params=(
            pltpu.CompilerParams(collective_id=0) if comm_on
            else pltpu.CompilerParams()
        ),
    )(x, Wq, Wo, Wk, Wv)


# device time: 23516 ns/iter; 1.6828x vs baseline; 1.0086x over previous
import os

import jax
import jax.numpy as jnp
from jax import lax
from jax.experimental import pallas as pl
from jax.experimental.pallas import tpu as pltpu

N_DEV = 32
DH = 64


def kernel(x, Wq, Wo, Wk, Wv):
    B, Sq, D = x.shape
    Hd = Wq.shape[1]
    Hq = Hd // DH
    R = B * Sq
    P = R // N_DEV
    bf16 = jnp.bfloat16
    comm_on = os.environ.get("ABLATE_COMM") != "1"

    def body(x_ref, wq_ref, wo_ref, wk_ref, wv_ref, out_ref,
             o_ref, acc_ref, s_ref, sbuf_ref, rs_ref, agbuf_ref,
             rs_send_sem, rs_recv_sem, ag_send_sem, ag_recv_sem):
        my_pos = lax.axis_index("i")

        if comm_on:
            barrier_sem = pltpu.get_barrier_semaphore()
            for j in range(N_DEV):
                pl.semaphore_signal(
                    barrier_sem, inc=1, device_id=j,
                    device_id_type=pl.DeviceIdType.LOGICAL,
                )

        x2 = x_ref[:].reshape(R, D).astype(bf16)
        qf = jnp.dot(x2, wq_ref[:].astype(bf16),
                     preferred_element_type=jnp.float32).astype(bf16)
        kf = jnp.dot(x2, wk_ref[:].astype(bf16),
                     preferred_element_type=jnp.float32).astype(bf16)
        vf = jnp.dot(x2, wv_ref[:].astype(bf16),
                     preferred_element_type=jnp.float32).astype(bf16)

        def compute_batch(b):
            rows = slice(b * Sq, (b + 1) * Sq)
            for h in range(Hq):
                cols = slice(h * DH, (h + 1) * DH)
                s_ref[h] = lax.dot_general(
                    qf[rows, cols], kf[rows, cols], (((1,), (1,)), ((), ())),
                    preferred_element_type=jnp.float32,
                )
            s_all = s_ref[:] * 0.125
            m = jnp.max(s_all, axis=-1, keepdims=True)
            p_ = jnp.exp(s_all - m)
            l_ = jnp.sum(p_, axis=-1, keepdims=True)
            pn = (p_ / l_).astype(bf16)
            for h in range(Hq):
                cols = slice(h * DH, (h + 1) * DH)
                o_ref[b, :, cols] = jnp.dot(
                    pn[h], vf[rows, cols], preferred_element_type=jnp.float32
                ).astype(bf16)
            acc_ref[b] = jnp.dot(
                o_ref[b], wo_ref[:].astype(bf16),
                preferred_element_type=jnp.float32,
            )

        def rs_send(j):
            return pltpu.make_async_remote_copy(
                src_ref=sbuf_ref.at[j],
                dst_ref=rs_ref.at[my_pos],
                send_sem=rs_send_sem,
                recv_sem=rs_recv_sem,
                device_id=j,
                device_id_type=pl.DeviceIdType.LOGICAL,
            )

        compute_batch(0)
        if comm_on:
            pl.semaphore_wait(barrier_sem, N_DEV)
            sbuf_ref[0:N_DEV // 2] = acc_ref[0].astype(bf16).reshape(
                N_DEV // 2, P, D)
            for j in range(N_DEV // 2):
                rs_send(j).start()
        compute_batch(1)

        if not comm_on:
            out_ref[:] = acc_ref[:].astype(bf16)
            return

        sbuf_ref[N_DEV // 2:N_DEV] = acc_ref[1].astype(bf16).reshape(
            N_DEV // 2, P, D)
        for j in range(N_DEV // 2, N_DEV):
            rs_send(j).start()

        for _ in range(N_DEV):
            rs_send(0).wait_recv()
        agbuf_ref[:] = jnp.sum(
            rs_ref[:].astype(jnp.float32), axis=0).astype(bf16)

        ppb = N_DEV // B
        b_idx = my_pos // ppb
        r0 = (my_pos % ppb) * P

        def ag_send(j):
            return pltpu.make_async_remote_copy(
                src_ref=agbuf_ref,
                dst_ref=out_ref.at[b_idx, pl.ds(r0, P)],
                send_sem=ag_send_sem,
                recv_sem=ag_recv_sem,
                device_id=j,
                device_id_type=pl.DeviceIdType.LOGICAL,
            )

        for j in range(N_DEV):
            ag_send(j).start()
        for _ in range(N_DEV):
            ag_send(0).wait_recv()

        for _ in range(N_DEV):
            rs_send(0).wait_send()
            ag_send(0).wait_send()

    return pl.pallas_call(
        body,
        out_shape=jax.ShapeDtypeStruct((B, Sq, D), bf16),
        in_specs=[pl.BlockSpec(memory_space=pltpu.VMEM)] * 5,
        out_specs=pl.BlockSpec(memory_space=pltpu.VMEM),
        scratch_shapes=[
            pltpu.VMEM((B, Sq, Hd), bf16),
            pltpu.VMEM((B, Sq, D), jnp.float32),
            pltpu.VMEM((Hq, Sq, Sq), jnp.float32),
            pltpu.VMEM((N_DEV, P, D), bf16),
            pltpu.VMEM((N_DEV, P, D), bf16),
            pltpu.VMEM((P, D), bf16),
            pltpu.SemaphoreType.DMA,
            pltpu.SemaphoreType.DMA,
            pltpu.SemaphoreType.DMA,
            pltpu.SemaphoreType.DMA,
        ],
        compiler_params=(
            pltpu.CompilerParams(collective_id=0) if comm_on
            else pltpu.CompilerParams()
        ),
    )(x, Wq, Wo, Wk, Wv)
